# baseline (speedup 1.0000x reference)
import functools
import math

import jax
import jax.numpy as jnp
from jax import lax
from jax.experimental import pallas as pl
from jax.experimental.pallas import tpu as pltpu
from jax.experimental.pallas import tpu_sc as plsc

N = 100000
E = 1600000
NG = 16
NS = 9
NRB = 8
HID = 64
RC = 10.0
PP = 6
AVG = 16.0

NC = 2
NSUB = 16
NW = NC * NSUB
L = 16

G_PAD = 12544
E_PAD = G_PAD * 128
EQ = E_PAD // 4
CH1 = 4096
NCH1 = EQ // CH1

GPW = G_PAD // NW
CHG = 28
NCHUNK = GPW // CHG

N_PAD = 100352
APT = N_PAD // NSUB

BLK = 8192
GRID = E_PAD // BLK


def _stage1_body(t4_hbm, sr_hbm, g8_hbm, tabv, idxv, outv, sem):
    cid = lax.axis_index("c")
    sid = lax.axis_index("s")
    wid = sid * NC + cid
    ep = wid // 16
    rem = wid % 16
    coord = rem // 4
    rng = rem % 4
    row = ep * 4 + coord
    ebase = rng * EQ
    pltpu.sync_copy(t4_hbm.at[coord], tabv)

    def chunk(k, carry):
        eb = ebase + k * CH1
        pltpu.sync_copy(sr_hbm.at[ep, pl.ds(eb, CH1)], idxv)

        def vreg(i, c2):
            sl = pl.ds(i * L, L)
            outv[sl] = plsc.load_gather(tabv, [idxv[sl]])
            return c2

        lax.fori_loop(0, CH1 // L, vreg, 0)
        pltpu.sync_copy(outv, g8_hbm.at[row, pl.ds(eb, CH1)])
        return carry

    lax.fori_loop(0, NCH1, chunk, 0)


def _gather_stage(t4, sr):
    f32 = jnp.float32
    return pl.kernel(
        _stage1_body,
        out_type=jax.ShapeDtypeStruct((8, E_PAD), f32),
        mesh=plsc.VectorSubcoreMesh(core_axis_name="c", subcore_axis_name="s"),
        compiler_params=pltpu.CompilerParams(use_tc_tiling_on_sc=False,
                                             needs_layout_passes=False),
        scratch_types=[
            pltpu.VMEM((N_PAD,), f32),
            pltpu.VMEM((CH1,), jnp.int32),
            pltpu.VMEM((CH1,), f32),
            pltpu.SemaphoreType.DMA,
        ],
    )(t4, sr)


def _mlp_body(g_ref, w1t_ref, w2t_ref, wot_ref, p_ref, o_ref):
    f32 = jnp.float32
    g = g_ref[...]
    dif = g[4:7, :] - g[0:3, :]
    d2 = jnp.sum(dif * dif, axis=0, keepdims=True)
    ss = g[3:4, :]
    sr = g[7:8, :]
    d = jnp.sqrt(d2 + 1e-12)
    x = d * (1.0 / RC)
    x2 = x * x
    x3 = x2 * x
    x6 = x3 * x3
    x7 = x6 * x
    x8 = x7 * x
    p = float(PP)
    env = (1.0 - ((p + 1.0) * (p + 2.0) / 2.0) * x6
           + p * (p + 2.0) * x7
           - (p * (p + 1.0) / 2.0) * x8)
    env = jnp.where(x < 1.0, env, 0.0)
    nvec = (lax.broadcasted_iota(jnp.int32, (NRB, 1), 0) + 1).astype(f32)
    s = jnp.sin(nvec * jnp.pi * x)
    rb = (math.sqrt(2.0 / RC) / (d + 1e-8) * env) * s
    i9 = lax.broadcasted_iota(jnp.int32, (NS, 1), 0).astype(f32)
    os_ = (ss == i9).astype(f32)
    orr = (sr == i9).astype(f32)
    feat = jnp.concatenate([rb, os_, orr], axis=0)
    bf16 = jnp.bfloat16
    dot = lambda a, b: lax.dot_general(
        a, b, (((1,), (0,)), ((), ())), preferred_element_type=f32)
    silu = lambda v: (0.5 * v) * jnp.tanh(0.5 * v) + (0.5 * v)
    h = dot(w1t_ref[...], feat.astype(bf16))
    h = silu(h)
    h = dot(w2t_ref[...], h.astype(bf16))
    h = silu(h)
    e = dot(wot_ref[...], h)
    ps = p_ref[...] @ os_
    pair = jnp.sum(ps * orr, axis=0, keepdims=True)
    o_ref[...] = e * pair * (1.0 / math.sqrt(AVG))


def _mlp_stage(g8, w1, w2, wo, pmat):
    f32 = jnp.float32
    gspec = pl.BlockSpec((8, BLK), lambda i: (0, i))
    ospec = pl.BlockSpec((1, BLK), lambda i: (0, i))
    wspec = lambda shape: pl.BlockSpec(shape, lambda i: (0, 0))
    return pl.pallas_call(
        _mlp_body,
        grid=(GRID,),
        in_specs=[gspec,
                  wspec((HID, NRB + 2 * NS)), wspec((HID, HID)),
                  wspec((1, HID)), wspec((NS, NS))],
        out_specs=ospec,
        out_shape=jax.ShapeDtypeStruct((1, E_PAD), f32),
    )(g8, w1.T.astype(jnp.bfloat16), w2.T.astype(jnp.bfloat16), wo.T, pmat)


def _stage3_body(s2d_hbm, v2d_hbm, z_hbm, g_hbm, stab_hbm, shtab_hbm, out_hbm,
                 sidx, vv, av, zv, gv, zerob, stab_v, shtab_v, bins_v, tmp16,
                 acc, sbins, sem):
    f32 = jnp.float32
    cid = lax.axis_index("c")
    sid = lax.axis_index("s")
    wid = sid * NC + cid
    iota16 = lax.iota(jnp.int32, L)

    def zloop(i, c):
        zerob[pl.ds(i * L, L)] = jnp.zeros((L,), f32)
        return c

    lax.fori_loop(0, APT // L, zloop, 0)
    pltpu.sync_copy(zerob, acc.at[pl.ds(sid * APT, APT)])

    @pl.when(sid == 0)
    def _():
        pltpu.sync_copy(zerob.at[pl.ds(0, L)], sbins)

    plsc.subcore_barrier()

    def chunk(k, carry):
        gb = wid * GPW + k * CHG
        pltpu.sync_copy(s2d_hbm.at[pl.ds(gb, CHG)], sidx)
        pltpu.sync_copy(v2d_hbm.at[pl.ds(gb, CHG)], vv)
        descs = []
        for j in range(CHG):
            descs.append(
                pltpu.async_copy(vv.at[j], acc.at[sidx.at[j]], sem, add=True))
        for dsc in descs:
            dsc.wait()
        return carry

    lax.fori_loop(0, NCHUNK, chunk, 0)
    plsc.subcore_barrier()

    pltpu.sync_copy(stab_hbm, stab_v)
    pltpu.sync_copy(shtab_hbm, shtab_v)
    shmul = jnp.where(cid == 0, 1.0, 0.0).astype(f32)
    bins_v[...] = jnp.zeros((L,), f32)
    ab = sid * APT
    pltpu.sync_copy(acc.at[pl.ds(ab, APT)], av)
    pltpu.sync_copy(z_hbm.at[pl.ds(ab, APT)], zv)
    pltpu.sync_copy(g_hbm.at[pl.ds(ab, APT)], gv)

    def vloop(i, carry):
        sl = pl.ds(i * L, L)
        z16 = zv[sl]
        g16 = gv[sl]
        sc16 = plsc.load_gather(stab_v, [z16])
        sh16 = plsc.load_gather(shtab_v, [z16])
        a = av[sl] * sc16 + sh16 * shmul
        plsc.addupdate_scatter(bins_v, [g16], a)
        return carry

    lax.fori_loop(0, APT // L, vloop, 0)
    pltpu.sync_copy(bins_v, sbins.at[iota16], add=True)
    plsc.subcore_barrier()

    @pl.when(sid == 0)
    def _():
        pltpu.sync_copy(sbins, tmp16)
        pltpu.sync_copy(tmp16, out_hbm.at[cid])


def _scatter_stage(s2d, v2d, z_pad, g_pad, stab, shtab):
    f32 = jnp.float32
    return pl.kernel(
        _stage3_body,
        out_type=jax.ShapeDtypeStruct((NC, L), f32),
        mesh=plsc.VectorSubcoreMesh(core_axis_name="c", subcore_axis_name="s"),
        compiler_params=pltpu.CompilerParams(use_tc_tiling_on_sc=False,
                                             needs_layout_passes=False),
        scratch_types=[
            pltpu.VMEM((CHG, 128), jnp.int32),
            pltpu.VMEM((CHG, 128), f32),
            pltpu.VMEM((APT,), f32),
            pltpu.VMEM((APT,), jnp.int32),
            pltpu.VMEM((APT,), jnp.int32),
            pltpu.VMEM((APT,), f32),
            pltpu.VMEM((L,), f32),
            pltpu.VMEM((L,), f32),
            pltpu.VMEM((L,), f32),
            pltpu.VMEM((L,), f32),
            pltpu.VMEM_SHARED((N_PAD,), f32),
            pltpu.VMEM_SHARED((L,), f32),
            pltpu.SemaphoreType.DMA,
        ],
    )(s2d, v2d, z_pad, g_pad, stab, shtab)


def kernel(pos, z, senders, receivers, graph_idx, n_graphs,
           W1, W2, Wout, pair_scale_raw, species_scale_raw, species_shift):
    f32 = jnp.float32
    i32 = jnp.int32

    t4 = jnp.concatenate([pos.T, z.astype(f32)[None, :]], axis=0)
    t4 = jnp.pad(t4, ((0, 0), (0, N_PAD - N)))
    s_pad = jnp.pad(senders.astype(i32), (0, E_PAD - E), constant_values=N)
    r_pad = jnp.pad(receivers.astype(i32), (0, E_PAD - E), constant_values=N)
    sr = jnp.stack([s_pad, r_pad])
    s2d = s_pad.reshape(G_PAD, 128)
    z_pad = jnp.pad(z.astype(i32), (0, N_PAD - N), constant_values=NS + 6)
    g_pad = jnp.pad(graph_idx.astype(i32), (0, N_PAD - N))

    pmat = jax.nn.softplus((pair_scale_raw + pair_scale_raw.T) / 2.0)
    stab = jnp.zeros((L,), f32).at[:NS].set(jax.nn.softplus(species_scale_raw))
    shtab = jnp.zeros((L,), f32).at[:NS].set(species_shift)

    g8 = _gather_stage(t4, sr)

    scaled = _mlp_stage(g8, W1, W2, Wout, pmat)

    parts = _scatter_stage(s2d, scaled.reshape(G_PAD, 128), z_pad, g_pad,
                           stab, shtab)
    return parts[0] + parts[1]


# speedup vs baseline: 1.0163x; 1.0163x over previous
import functools
import math

import jax
import jax.numpy as jnp
from jax import lax
from jax.experimental import pallas as pl
from jax.experimental.pallas import tpu as pltpu
from jax.experimental.pallas import tpu_sc as plsc

N = 100000
E = 1600000
NG = 16
NS = 9
NRB = 8
HID = 64
RC = 10.0
PP = 6
AVG = 16.0

NC = 2
NSUB = 16
NW = NC * NSUB
L = 16

G_PAD = 12544
E_PAD = G_PAD * 128
EQ = E_PAD // 4
CH1 = 4096
NCH1 = EQ // CH1

GPW = G_PAD // NW
CHG = 28
NCHUNK = GPW // CHG

N_PAD = 100352
APT = N_PAD // NSUB

BLK = 8192
GRID = E_PAD // BLK


def _stage1_body(t4_hbm, sr_hbm, g8_hbm, tabv, idxv, outv,
                 si0, si1, so0, so1):
    cid = lax.axis_index("c")
    sid = lax.axis_index("s")
    wid = sid * NC + cid
    ep = wid // 16
    rem = wid % 16
    coord = rem // 4
    rng = rem % 4
    row = ep * 4 + coord
    ebase = rng * EQ
    sis = (si0, si1)
    sos = (so0, so1)

    def in_copy(k, par):
        return pltpu.make_async_copy(
            sr_hbm.at[ep, pl.ds(ebase + k * CH1, CH1)], idxv.at[par], sis[par])

    def out_copy(k, par):
        return pltpu.make_async_copy(
            outv.at[par], g8_hbm.at[row, pl.ds(ebase + k * CH1, CH1)],
            sos[par])

    pltpu.sync_copy(t4_hbm.at[coord], tabv)
    in_copy(0, 0).start()

    def pair(kk, carry):
        for par in range(2):
            k = 2 * kk + par
            in_copy(k, par).wait()

            @pl.when(k + 1 < NCH1)
            def _():
                in_copy(k + 1, 1 - par).start()

            @pl.when(kk > 0)
            def _():
                out_copy(k - 2, par).wait()

            def vreg(i, c2):
                sl = pl.ds(i * L, L)
                outv[par, sl] = plsc.load_gather(tabv, [idxv[par, sl]])
                return c2

            lax.fori_loop(0, CH1 // L, vreg, 0, unroll=8)
            out_copy(k, par).start()
        return carry

    lax.fori_loop(0, NCH1 // 2, pair, 0)
    out_copy(NCH1 - 2, 0).wait()
    out_copy(NCH1 - 1, 1).wait()


def _gather_stage(t4, sr):
    f32 = jnp.float32
    return pl.kernel(
        _stage1_body,
        out_type=jax.ShapeDtypeStruct((8, E_PAD), f32),
        mesh=plsc.VectorSubcoreMesh(core_axis_name="c", subcore_axis_name="s"),
        compiler_params=pltpu.CompilerParams(use_tc_tiling_on_sc=False,
                                             needs_layout_passes=False),
        scratch_types=[
            pltpu.VMEM((N_PAD,), f32),
            pltpu.VMEM((2, CH1), jnp.int32),
            pltpu.VMEM((2, CH1), f32),
            pltpu.SemaphoreType.DMA,
            pltpu.SemaphoreType.DMA,
            pltpu.SemaphoreType.DMA,
            pltpu.SemaphoreType.DMA,
        ],
    )(t4, sr)


def _mlp_body(g_ref, w1t_ref, w2t_ref, wot_ref, p_ref, o_ref):
    f32 = jnp.float32
    g = g_ref[...]
    dif = g[4:7, :] - g[0:3, :]
    d2 = jnp.sum(dif * dif, axis=0, keepdims=True)
    ss = g[3:4, :]
    sr = g[7:8, :]
    d = jnp.sqrt(d2 + 1e-12)
    x = d * (1.0 / RC)
    x2 = x * x
    x3 = x2 * x
    x6 = x3 * x3
    x7 = x6 * x
    x8 = x7 * x
    p = float(PP)
    env = (1.0 - ((p + 1.0) * (p + 2.0) / 2.0) * x6
           + p * (p + 2.0) * x7
           - (p * (p + 1.0) / 2.0) * x8)
    env = jnp.where(x < 1.0, env, 0.0)
    nvec = (lax.broadcasted_iota(jnp.int32, (NRB, 1), 0) + 1).astype(f32)
    s = jnp.sin(nvec * jnp.pi * x)
    rb = (math.sqrt(2.0 / RC) / (d + 1e-8) * env) * s
    i9 = lax.broadcasted_iota(jnp.int32, (NS, 1), 0).astype(f32)
    os_ = (ss == i9).astype(f32)
    orr = (sr == i9).astype(f32)
    feat = jnp.concatenate([rb, os_, orr], axis=0)
    bf16 = jnp.bfloat16
    dot = lambda a, b: lax.dot_general(
        a, b, (((1,), (0,)), ((), ())), preferred_element_type=f32)
    silu = lambda v: (0.5 * v) * jnp.tanh(0.5 * v) + (0.5 * v)
    h = dot(w1t_ref[...], feat.astype(bf16))
    h = silu(h)
    h = dot(w2t_ref[...], h.astype(bf16))
    h = silu(h)
    e = dot(wot_ref[...], h)
    ps = p_ref[...] @ os_
    pair = jnp.sum(ps * orr, axis=0, keepdims=True)
    o_ref[...] = e * pair * (1.0 / math.sqrt(AVG))


def _mlp_stage(g8, w1, w2, wo, pmat):
    f32 = jnp.float32
    gspec = pl.BlockSpec((8, BLK), lambda i: (0, i))
    ospec = pl.BlockSpec((1, BLK), lambda i: (0, i))
    wspec = lambda shape: pl.BlockSpec(shape, lambda i: (0, 0))
    return pl.pallas_call(
        _mlp_body,
        grid=(GRID,),
        in_specs=[gspec,
                  wspec((HID, NRB + 2 * NS)), wspec((HID, HID)),
                  wspec((1, HID)), wspec((NS, NS))],
        out_specs=ospec,
        out_shape=jax.ShapeDtypeStruct((1, E_PAD), f32),
    )(g8, w1.T.astype(jnp.bfloat16), w2.T.astype(jnp.bfloat16), wo.T, pmat)


def _stage3_body(s2d_hbm, v2d_hbm, z_hbm, g_hbm, stab_hbm, shtab_hbm, out_hbm,
                 sidx, vv, av, zv, gv, zerob, stab_v, shtab_v, bins_v, tmp16,
                 acc, sbins, sem):
    f32 = jnp.float32
    cid = lax.axis_index("c")
    sid = lax.axis_index("s")
    wid = sid * NC + cid
    iota16 = lax.iota(jnp.int32, L)

    def zloop(i, c):
        zerob[pl.ds(i * L, L)] = jnp.zeros((L,), f32)
        return c

    lax.fori_loop(0, APT // L, zloop, 0)
    pltpu.sync_copy(zerob, acc.at[pl.ds(sid * APT, APT)])

    @pl.when(sid == 0)
    def _():
        pltpu.sync_copy(zerob.at[pl.ds(0, L)], sbins)

    plsc.subcore_barrier()

    def chunk(k, carry):
        gb = wid * GPW + k * CHG
        pltpu.sync_copy(s2d_hbm.at[pl.ds(gb, CHG)], sidx)
        pltpu.sync_copy(v2d_hbm.at[pl.ds(gb, CHG)], vv)
        descs = []
        for j in range(CHG):
            descs.append(
                pltpu.async_copy(vv.at[j], acc.at[sidx.at[j]], sem, add=True))
        for dsc in descs:
            dsc.wait()
        return carry

    lax.fori_loop(0, NCHUNK, chunk, 0)
    plsc.subcore_barrier()

    pltpu.sync_copy(stab_hbm, stab_v)
    pltpu.sync_copy(shtab_hbm, shtab_v)
    shmul = jnp.where(cid == 0, 1.0, 0.0).astype(f32)
    bins_v[...] = jnp.zeros((L,), f32)
    ab = sid * APT
    pltpu.sync_copy(acc.at[pl.ds(ab, APT)], av)
    pltpu.sync_copy(z_hbm.at[pl.ds(ab, APT)], zv)
    pltpu.sync_copy(g_hbm.at[pl.ds(ab, APT)], gv)

    def vloop(i, carry):
        sl = pl.ds(i * L, L)
        z16 = zv[sl]
        g16 = gv[sl]
        sc16 = plsc.load_gather(stab_v, [z16])
        sh16 = plsc.load_gather(shtab_v, [z16])
        a = av[sl] * sc16 + sh16 * shmul
        plsc.addupdate_scatter(bins_v, [g16], a)
        return carry

    lax.fori_loop(0, APT // L, vloop, 0)
    pltpu.sync_copy(bins_v, sbins.at[iota16], add=True)
    plsc.subcore_barrier()

    @pl.when(sid == 0)
    def _():
        pltpu.sync_copy(sbins, tmp16)
        pltpu.sync_copy(tmp16, out_hbm.at[cid])


def _scatter_stage(s2d, v2d, z_pad, g_pad, stab, shtab):
    f32 = jnp.float32
    return pl.kernel(
        _stage3_body,
        out_type=jax.ShapeDtypeStruct((NC, L), f32),
        mesh=plsc.VectorSubcoreMesh(core_axis_name="c", subcore_axis_name="s"),
        compiler_params=pltpu.CompilerParams(use_tc_tiling_on_sc=False,
                                             needs_layout_passes=False),
        scratch_types=[
            pltpu.VMEM((CHG, 128), jnp.int32),
            pltpu.VMEM((CHG, 128), f32),
            pltpu.VMEM((APT,), f32),
            pltpu.VMEM((APT,), jnp.int32),
            pltpu.VMEM((APT,), jnp.int32),
            pltpu.VMEM((APT,), f32),
            pltpu.VMEM((L,), f32),
            pltpu.VMEM((L,), f32),
            pltpu.VMEM((L,), f32),
            pltpu.VMEM((L,), f32),
            pltpu.VMEM_SHARED((N_PAD,), f32),
            pltpu.VMEM_SHARED((L,), f32),
            pltpu.SemaphoreType.DMA,
        ],
    )(s2d, v2d, z_pad, g_pad, stab, shtab)


def kernel(pos, z, senders, receivers, graph_idx, n_graphs,
           W1, W2, Wout, pair_scale_raw, species_scale_raw, species_shift):
    f32 = jnp.float32
    i32 = jnp.int32

    t4 = jnp.concatenate([pos.T, z.astype(f32)[None, :]], axis=0)
    t4 = jnp.pad(t4, ((0, 0), (0, N_PAD - N)))
    s_pad = jnp.pad(senders.astype(i32), (0, E_PAD - E), constant_values=N)
    r_pad = jnp.pad(receivers.astype(i32), (0, E_PAD - E), constant_values=N)
    sr = jnp.stack([s_pad, r_pad])
    s2d = s_pad.reshape(G_PAD, 128)
    z_pad = jnp.pad(z.astype(i32), (0, N_PAD - N), constant_values=NS + 6)
    g_pad = jnp.pad(graph_idx.astype(i32), (0, N_PAD - N))

    pmat = jax.nn.softplus((pair_scale_raw + pair_scale_raw.T) / 2.0)
    stab = jnp.zeros((L,), f32).at[:NS].set(jax.nn.softplus(species_scale_raw))
    shtab = jnp.zeros((L,), f32).at[:NS].set(species_shift)

    g8 = _gather_stage(t4, sr)

    scaled = _mlp_stage(g8, W1, W2, Wout, pmat)

    parts = _scatter_stage(s2d, scaled.reshape(G_PAD, 128), z_pad, g_pad,
                           stab, shtab)
    return parts[0] + parts[1]


# speedup vs baseline: 1.0262x; 1.0097x over previous
import functools
import math

import jax
import jax.numpy as jnp
from jax import lax
from jax.experimental import pallas as pl
from jax.experimental.pallas import tpu as pltpu
from jax.experimental.pallas import tpu_sc as plsc

N = 100000
E = 1600000
NG = 16
NS = 9
NRB = 8
HID = 64
RC = 10.0
PP = 6
AVG = 16.0

NC = 2
NSUB = 16
NW = NC * NSUB
L = 16

G_PAD = 12544
E_PAD = G_PAD * 128
EQ = E_PAD // 4
CH1 = 4096
NCH1 = EQ // CH1

GPW = G_PAD // NW
CHG = 28
NCHUNK = GPW // CHG
CH3 = CHG * 128

N_PAD = 100352
APT = N_PAD // NSUB

BLK = 8192
GRID = E_PAD // BLK


def _stage1_body(t4_hbm, s_hbm, r_hbm, g8_hbm, tabv, idxv, outv,
                 si0, si1, so0, so1):
    cid = lax.axis_index("c")
    sid = lax.axis_index("s")
    wid = sid * NC + cid
    ep = wid // 16
    rem = wid % 16
    coord = rem // 4
    rng = rem % 4
    row = ep * 4 + coord
    ebase = rng * EQ
    sis = (si0, si1)
    sos = (so0, so1)

    def in_copy(k, par, src):
        return pltpu.make_async_copy(
            src.at[pl.ds(ebase + k * CH1, CH1)], idxv.at[par], sis[par])

    def in_start(k, par):
        @pl.when(ep == 0)
        def _():
            in_copy(k, par, s_hbm).start()

        @pl.when(ep == 1)
        def _():
            in_copy(k, par, r_hbm).start()

    def out_copy(k, par):
        return pltpu.make_async_copy(
            outv.at[par],
            g8_hbm.at[pl.ds(row * E_PAD + ebase + k * CH1, CH1)], sos[par])

    pltpu.sync_copy(t4_hbm.at[pl.ds(coord * N_PAD, N_PAD)], tabv)
    in_start(0, 0)

    def pair(kk, carry):
        for par in range(2):
            k = 2 * kk + par
            in_copy(k, par, s_hbm).wait()

            @pl.when(k + 1 < NCH1)
            def _():
                in_start(k + 1, 1 - par)

            @pl.when(kk > 0)
            def _():
                out_copy(k - 2, par).wait()

            def vreg(i, c2):
                sl = pl.ds(i * L, L)
                outv[par, sl] = plsc.load_gather(tabv, [idxv[par, sl]])
                return c2

            lax.fori_loop(0, CH1 // L, vreg, 0, unroll=8)
            out_copy(k, par).start()
        return carry

    lax.fori_loop(0, NCH1 // 2, pair, 0)
    out_copy(NCH1 - 2, 0).wait()
    out_copy(NCH1 - 1, 1).wait()


def _gather_stage(t4, s_pad, r_pad):
    f32 = jnp.float32
    return pl.kernel(
        _stage1_body,
        out_type=jax.ShapeDtypeStruct((8 * E_PAD,), f32),
        mesh=plsc.VectorSubcoreMesh(core_axis_name="c", subcore_axis_name="s"),
        compiler_params=pltpu.CompilerParams(use_tc_tiling_on_sc=False,
                                             needs_layout_passes=False),
        scratch_types=[
            pltpu.VMEM((N_PAD,), f32),
            pltpu.VMEM((2, CH1), jnp.int32),
            pltpu.VMEM((2, CH1), f32),
            pltpu.SemaphoreType.DMA,
            pltpu.SemaphoreType.DMA,
            pltpu.SemaphoreType.DMA,
            pltpu.SemaphoreType.DMA,
        ],
    )(t4, s_pad, r_pad)


def _mlp_body(g_ref, w1t_ref, w2t_ref, wot_ref, p_ref, o_ref):
    f32 = jnp.float32
    g = g_ref[...]
    dif = g[4:7, :] - g[0:3, :]
    d2 = jnp.sum(dif * dif, axis=0, keepdims=True)
    ss = g[3:4, :]
    sr = g[7:8, :]
    d = jnp.sqrt(d2 + 1e-12)
    x = d * (1.0 / RC)
    x2 = x * x
    x3 = x2 * x
    x6 = x3 * x3
    x7 = x6 * x
    x8 = x7 * x
    p = float(PP)
    env = (1.0 - ((p + 1.0) * (p + 2.0) / 2.0) * x6
           + p * (p + 2.0) * x7
           - (p * (p + 1.0) / 2.0) * x8)
    env = jnp.where(x < 1.0, env, 0.0)
    nvec = (lax.broadcasted_iota(jnp.int32, (NRB, 1), 0) + 1).astype(f32)
    s = jnp.sin(nvec * jnp.pi * x)
    rb = (math.sqrt(2.0 / RC) / (d + 1e-8) * env) * s
    i9 = lax.broadcasted_iota(jnp.int32, (NS, 1), 0).astype(f32)
    os_ = (ss == i9).astype(f32)
    orr = (sr == i9).astype(f32)
    feat = jnp.concatenate([rb, os_, orr], axis=0)
    bf16 = jnp.bfloat16
    dot = lambda a, b: lax.dot_general(
        a, b, (((1,), (0,)), ((), ())), preferred_element_type=f32)
    silu = lambda v: (0.5 * v) * jnp.tanh(0.5 * v) + (0.5 * v)
    h = dot(w1t_ref[...], feat.astype(bf16))
    h = silu(h)
    h = dot(w2t_ref[...], h.astype(bf16))
    h = silu(h)
    e = dot(wot_ref[...], h)
    ps = p_ref[...] @ os_
    pair = jnp.sum(ps * orr, axis=0, keepdims=True)
    o_ref[...] = (e * pair * (1.0 / math.sqrt(AVG))).reshape(BLK)


def _mlp_stage(g8, w1, w2, wo, pmat):
    f32 = jnp.float32
    gspec = pl.BlockSpec((8, BLK), lambda i: (0, i))
    ospec = pl.BlockSpec((BLK,), lambda i: (i,))
    wspec = lambda shape: pl.BlockSpec(shape, lambda i: (0, 0))
    return pl.pallas_call(
        _mlp_body,
        grid=(GRID,),
        in_specs=[gspec,
                  wspec((HID, NRB + 2 * NS)), wspec((HID, HID)),
                  wspec((1, HID)), wspec((NS, NS))],
        out_specs=ospec,
        out_shape=jax.ShapeDtypeStruct((E_PAD,), f32),
    )(g8, w1.T.astype(jnp.bfloat16), w2.T.astype(jnp.bfloat16), wo.T, pmat)


def _stage3_body(s_hbm, v_hbm, z_hbm, g_hbm, stab_hbm, shtab_hbm, out_hbm,
                 sidx, vv, av, zv, gv, zerob, stab_v, shtab_v, bins_v, tmp16,
                 acc, sbins, sem):
    f32 = jnp.float32
    cid = lax.axis_index("c")
    sid = lax.axis_index("s")
    wid = sid * NC + cid
    iota16 = lax.iota(jnp.int32, L)

    def zloop(i, c):
        zerob[pl.ds(i * L, L)] = jnp.zeros((L,), f32)
        return c

    lax.fori_loop(0, APT // L, zloop, 0)
    pltpu.sync_copy(zerob, acc.at[pl.ds(sid * APT, APT)])

    @pl.when(sid == 0)
    def _():
        pltpu.sync_copy(zerob.at[pl.ds(0, L)], sbins)

    plsc.subcore_barrier()

    def chunk(k, carry):
        eb = (wid * GPW + k * CHG) * 128
        pltpu.sync_copy(s_hbm.at[pl.ds(eb, CH3)], sidx)
        pltpu.sync_copy(v_hbm.at[pl.ds(eb, CH3)], vv)
        descs = []
        for j in range(CHG):
            sl = pl.ds(j * 128, 128)
            descs.append(
                pltpu.async_copy(vv.at[sl], acc.at[sidx.at[sl]], sem,
                                 add=True))
        for dsc in descs:
            dsc.wait()
        return carry

    lax.fori_loop(0, NCHUNK, chunk, 0)
    plsc.subcore_barrier()

    pltpu.sync_copy(stab_hbm, stab_v)
    pltpu.sync_copy(shtab_hbm, shtab_v)
    shmul = jnp.where(cid == 0, 1.0, 0.0).astype(f32)
    bins_v[...] = jnp.zeros((L,), f32)
    ab = sid * APT
    pltpu.sync_copy(acc.at[pl.ds(ab, APT)], av)
    pltpu.sync_copy(z_hbm.at[pl.ds(ab, APT)], zv)
    pltpu.sync_copy(g_hbm.at[pl.ds(ab, APT)], gv)

    def vloop(i, carry):
        sl = pl.ds(i * L, L)
        z16 = zv[sl]
        g16 = gv[sl]
        sc16 = plsc.load_gather(stab_v, [z16])
        sh16 = plsc.load_gather(shtab_v, [z16])
        a = av[sl] * sc16 + sh16 * shmul
        plsc.addupdate_scatter(bins_v, [g16], a)
        return carry

    lax.fori_loop(0, APT // L, vloop, 0)
    pltpu.sync_copy(bins_v, sbins.at[iota16], add=True)
    plsc.subcore_barrier()

    @pl.when(sid == 0)
    def _():
        pltpu.sync_copy(sbins, tmp16)
        pltpu.sync_copy(tmp16, out_hbm.at[cid])


def _scatter_stage(s_pad, vals, z_pad, g_pad, stab, shtab):
    f32 = jnp.float32
    return pl.kernel(
        _stage3_body,
        out_type=jax.ShapeDtypeStruct((NC, L), f32),
        mesh=plsc.VectorSubcoreMesh(core_axis_name="c", subcore_axis_name="s"),
        compiler_params=pltpu.CompilerParams(use_tc_tiling_on_sc=False,
                                             needs_layout_passes=False),
        scratch_types=[
            pltpu.VMEM((CH3,), jnp.int32),
            pltpu.VMEM((CH3,), f32),
            pltpu.VMEM((APT,), f32),
            pltpu.VMEM((APT,), jnp.int32),
            pltpu.VMEM((APT,), jnp.int32),
            pltpu.VMEM((APT,), f32),
            pltpu.VMEM((L,), f32),
            pltpu.VMEM((L,), f32),
            pltpu.VMEM((L,), f32),
            pltpu.VMEM((L,), f32),
            pltpu.VMEM_SHARED((N_PAD,), f32),
            pltpu.VMEM_SHARED((L,), f32),
            pltpu.SemaphoreType.DMA,
        ],
    )(s_pad, vals, z_pad, g_pad, stab, shtab)


def kernel(pos, z, senders, receivers, graph_idx, n_graphs,
           W1, W2, Wout, pair_scale_raw, species_scale_raw, species_shift):
    f32 = jnp.float32
    i32 = jnp.int32

    t4 = jnp.concatenate([pos.T, z.astype(f32)[None, :]], axis=0)
    t4 = jnp.pad(t4, ((0, 0), (0, N_PAD - N))).reshape(-1)
    s_pad = jnp.pad(senders.astype(i32), (0, E_PAD - E), constant_values=N)
    r_pad = jnp.pad(receivers.astype(i32), (0, E_PAD - E), constant_values=N)
    z_pad = jnp.pad(z.astype(i32), (0, N_PAD - N), constant_values=NS + 6)
    g_pad = jnp.pad(graph_idx.astype(i32), (0, N_PAD - N))

    pmat = jax.nn.softplus((pair_scale_raw + pair_scale_raw.T) / 2.0)
    stab = jnp.zeros((L,), f32).at[:NS].set(jax.nn.softplus(species_scale_raw))
    shtab = jnp.zeros((L,), f32).at[:NS].set(species_shift)

    g8 = _gather_stage(t4, s_pad, r_pad)

    scaled = _mlp_stage(g8.reshape(8, E_PAD), W1, W2, Wout, pmat)

    parts = _scatter_stage(s_pad, scaled, z_pad, g_pad, stab, shtab)
    return parts[0] + parts[1]


# speedup vs baseline: 2.7777x; 2.7068x over previous
import functools
import math

import jax
import jax.numpy as jnp
from jax import lax
from jax.experimental import pallas as pl
from jax.experimental.pallas import tpu as pltpu
from jax.experimental.pallas import tpu_sc as plsc

N = 100000
E = 1600000
NG = 16
NS = 9
NRB = 8
HID = 64
RC = 10.0
PP = 6
AVG = 16.0

NC = 2
NSUB = 16
NW = NC * NSUB
L = 16

G_PAD = 12544
E_PAD = G_PAD * 128
EQ = E_PAD // 4
CH1 = 4096
NCH1 = EQ // CH1

GPW = G_PAD // NW
CHG = 28
NCHUNK = GPW // CHG
CH3 = CHG * 128

N_PAD = 100352
APT = N_PAD // NSUB

BLK = 8192
GRID = E_PAD // BLK


def _stage1_body(t4_hbm, s_hbm, r_hbm, g8_hbm, tabv, idxv, outv,
                 si0, si1, so0, so1):
    cid = lax.axis_index("c")
    sid = lax.axis_index("s")
    wid = sid * NC + cid
    ep = wid // 16
    rem = wid % 16
    coord = rem // 4
    rng = rem % 4
    row = ep * 4 + coord
    ebase = rng * EQ
    sis = (si0, si1)
    sos = (so0, so1)

    def in_copy(k, par, src):
        return pltpu.make_async_copy(
            src.at[pl.ds(ebase + k * CH1, CH1)], idxv.at[par], sis[par])

    def in_start(k, par):
        @pl.when(ep == 0)
        def _():
            in_copy(k, par, s_hbm).start()

        @pl.when(ep == 1)
        def _():
            in_copy(k, par, r_hbm).start()

    def out_copy(k, par):
        return pltpu.make_async_copy(
            outv.at[par],
            g8_hbm.at[pl.ds(row * E_PAD + ebase + k * CH1, CH1)], sos[par])

    pltpu.sync_copy(t4_hbm.at[pl.ds(coord * N_PAD, N_PAD)], tabv)
    in_start(0, 0)

    def pair(kk, carry):
        for par in range(2):
            k = 2 * kk + par
            in_copy(k, par, s_hbm).wait()

            @pl.when(k + 1 < NCH1)
            def _():
                in_start(k + 1, 1 - par)

            @pl.when(kk > 0)
            def _():
                out_copy(k - 2, par).wait()

            def vreg(i, c2):
                sl = pl.ds(i * L, L)
                outv[par, sl] = plsc.load_gather(tabv, [idxv[par, sl]])
                return c2

            lax.fori_loop(0, CH1 // L, vreg, 0, unroll=8)
            out_copy(k, par).start()
        return carry

    lax.fori_loop(0, NCH1 // 2, pair, 0)
    out_copy(NCH1 - 2, 0).wait()
    out_copy(NCH1 - 1, 1).wait()


def _gather_stage(t4, s_pad, r_pad):
    f32 = jnp.float32
    return pl.kernel(
        _stage1_body,
        out_type=jax.ShapeDtypeStruct((8 * E_PAD,), f32),
        mesh=plsc.VectorSubcoreMesh(core_axis_name="c", subcore_axis_name="s"),
        compiler_params=pltpu.CompilerParams(use_tc_tiling_on_sc=False,
                                             needs_layout_passes=False),
        scratch_types=[
            pltpu.VMEM((N_PAD,), f32),
            pltpu.VMEM((2, CH1), jnp.int32),
            pltpu.VMEM((2, CH1), f32),
            pltpu.SemaphoreType.DMA,
            pltpu.SemaphoreType.DMA,
            pltpu.SemaphoreType.DMA,
            pltpu.SemaphoreType.DMA,
        ],
    )(t4, s_pad, r_pad)


def _mlp_body(sx_ref, sy_ref, sz_ref, ssp_ref, rx_ref, ry_ref, rz_ref,
              rsp_ref, w1t_ref, w2t_ref, wot_ref, p_ref, o_ref):
    f32 = jnp.float32
    dx = rx_ref[...] - sx_ref[...]
    dy = ry_ref[...] - sy_ref[...]
    dz = rz_ref[...] - sz_ref[...]
    d2 = (dx * dx + dy * dy + dz * dz).reshape(1, BLK)
    ss = ssp_ref[...].reshape(1, BLK)
    sr = rsp_ref[...].reshape(1, BLK)
    d = jnp.sqrt(d2 + 1e-12)
    x = d * (1.0 / RC)
    x2 = x * x
    x3 = x2 * x
    x6 = x3 * x3
    x7 = x6 * x
    x8 = x7 * x
    p = float(PP)
    env = (1.0 - ((p + 1.0) * (p + 2.0) / 2.0) * x6
           + p * (p + 2.0) * x7
           - (p * (p + 1.0) / 2.0) * x8)
    env = jnp.where(x < 1.0, env, 0.0)
    nvec = (lax.broadcasted_iota(jnp.int32, (NRB, 1), 0) + 1).astype(f32)
    s = jnp.sin(nvec * jnp.pi * x)
    rb = (math.sqrt(2.0 / RC) / (d + 1e-8) * env) * s
    i9 = lax.broadcasted_iota(jnp.int32, (NS, 1), 0).astype(f32)
    os_ = (ss == i9).astype(f32)
    orr = (sr == i9).astype(f32)
    feat = jnp.concatenate([rb, os_, orr], axis=0)
    bf16 = jnp.bfloat16
    dot = lambda a, b: lax.dot_general(
        a, b, (((1,), (0,)), ((), ())), preferred_element_type=f32)
    silu = lambda v: (0.5 * v) * jnp.tanh(0.5 * v) + (0.5 * v)
    h = dot(w1t_ref[...], feat.astype(bf16))
    h = silu(h)
    h = dot(w2t_ref[...], h.astype(bf16))
    h = silu(h)
    e = dot(wot_ref[...], h)
    ps = p_ref[...] @ os_
    pair = jnp.sum(ps * orr, axis=0, keepdims=True)
    o_ref[...] = (e * pair * (1.0 / math.sqrt(AVG))).reshape(BLK)


def _mlp_stage(g8, w1, w2, wo, pmat):
    f32 = jnp.float32
    rspec = lambda r: pl.BlockSpec((BLK,), lambda i, r=r: (r * GRID + i,))
    ospec = pl.BlockSpec((BLK,), lambda i: (i,))
    wspec = lambda shape: pl.BlockSpec(shape, lambda i: (0, 0))
    return pl.pallas_call(
        _mlp_body,
        grid=(GRID,),
        in_specs=[rspec(r) for r in range(8)] +
                 [wspec((HID, NRB + 2 * NS)), wspec((HID, HID)),
                  wspec((1, HID)), wspec((NS, NS))],
        out_specs=ospec,
        out_shape=jax.ShapeDtypeStruct((E_PAD,), f32),
    )(*([g8] * 8), w1.T.astype(jnp.bfloat16), w2.T.astype(jnp.bfloat16),
      wo.T, pmat)


def _stage3_body(s_hbm, v_hbm, z_hbm, g_hbm, stab_hbm, shtab_hbm, out_hbm,
                 sidx, vv, av, zv, gv, zerob, stab_v, shtab_v, bins_v, tmp16,
                 acc, sbins, sem):
    f32 = jnp.float32
    cid = lax.axis_index("c")
    sid = lax.axis_index("s")
    wid = sid * NC + cid
    iota16 = lax.iota(jnp.int32, L)

    def zloop(i, c):
        zerob[pl.ds(i * L, L)] = jnp.zeros((L,), f32)
        return c

    lax.fori_loop(0, APT // L, zloop, 0)
    pltpu.sync_copy(zerob, acc.at[pl.ds(sid * APT, APT)])

    @pl.when(sid == 0)
    def _():
        pltpu.sync_copy(zerob.at[pl.ds(0, L)], sbins)

    plsc.subcore_barrier()

    def chunk(k, carry):
        eb = (wid * GPW + k * CHG) * 128
        pltpu.sync_copy(s_hbm.at[pl.ds(eb, CH3)], sidx)
        pltpu.sync_copy(v_hbm.at[pl.ds(eb, CH3)], vv)
        descs = []
        for j in range(CHG):
            sl = pl.ds(j * 128, 128)
            descs.append(
                pltpu.async_copy(vv.at[sl], acc.at[sidx.at[sl]], sem,
                                 add=True))
        for dsc in descs:
            dsc.wait()
        return carry

    lax.fori_loop(0, NCHUNK, chunk, 0)
    plsc.subcore_barrier()

    pltpu.sync_copy(stab_hbm, stab_v)
    pltpu.sync_copy(shtab_hbm, shtab_v)
    shmul = jnp.where(cid == 0, 1.0, 0.0).astype(f32)
    bins_v[...] = jnp.zeros((L,), f32)
    ab = sid * APT
    pltpu.sync_copy(acc.at[pl.ds(ab, APT)], av)
    pltpu.sync_copy(z_hbm.at[pl.ds(ab, APT)], zv)
    pltpu.sync_copy(g_hbm.at[pl.ds(ab, APT)], gv)

    def vloop(i, carry):
        sl = pl.ds(i * L, L)
        z16 = zv[sl]
        g16 = gv[sl]
        sc16 = plsc.load_gather(stab_v, [z16])
        sh16 = plsc.load_gather(shtab_v, [z16])
        a = av[sl] * sc16 + sh16 * shmul
        plsc.addupdate_scatter(bins_v, [g16], a)
        return carry

    lax.fori_loop(0, APT // L, vloop, 0)
    pltpu.sync_copy(bins_v, sbins.at[iota16], add=True)
    plsc.subcore_barrier()

    @pl.when(sid == 0)
    def _():
        pltpu.sync_copy(sbins, tmp16)
        pltpu.sync_copy(tmp16, out_hbm.at[cid])


def _scatter_stage(s_pad, vals, z_pad, g_pad, stab, shtab):
    f32 = jnp.float32
    return pl.kernel(
        _stage3_body,
        out_type=jax.ShapeDtypeStruct((NC, L), f32),
        mesh=plsc.VectorSubcoreMesh(core_axis_name="c", subcore_axis_name="s"),
        compiler_params=pltpu.CompilerParams(use_tc_tiling_on_sc=False,
                                             needs_layout_passes=False),
        scratch_types=[
            pltpu.VMEM((CH3,), jnp.int32),
            pltpu.VMEM((CH3,), f32),
            pltpu.VMEM((APT,), f32),
            pltpu.VMEM((APT,), jnp.int32),
            pltpu.VMEM((APT,), jnp.int32),
            pltpu.VMEM((APT,), f32),
            pltpu.VMEM((L,), f32),
            pltpu.VMEM((L,), f32),
            pltpu.VMEM((L,), f32),
            pltpu.VMEM((L,), f32),
            pltpu.VMEM_SHARED((N_PAD,), f32),
            pltpu.VMEM_SHARED((L,), f32),
            pltpu.SemaphoreType.DMA,
        ],
    )(s_pad, vals, z_pad, g_pad, stab, shtab)


def kernel(pos, z, senders, receivers, graph_idx, n_graphs,
           W1, W2, Wout, pair_scale_raw, species_scale_raw, species_shift):
    f32 = jnp.float32
    i32 = jnp.int32

    t4 = jnp.concatenate([pos.T, z.astype(f32)[None, :]], axis=0)
    t4 = jnp.pad(t4, ((0, 0), (0, N_PAD - N))).reshape(-1)
    s_pad = jnp.pad(senders.astype(i32), (0, E_PAD - E), constant_values=N)
    r_pad = jnp.pad(receivers.astype(i32), (0, E_PAD - E), constant_values=N)
    z_pad = jnp.pad(z.astype(i32), (0, N_PAD - N), constant_values=NS + 6)
    g_pad = jnp.pad(graph_idx.astype(i32), (0, N_PAD - N))

    pmat = jax.nn.softplus((pair_scale_raw + pair_scale_raw.T) / 2.0)
    stab = jnp.zeros((L,), f32).at[:NS].set(jax.nn.softplus(species_scale_raw))
    shtab = jnp.zeros((L,), f32).at[:NS].set(species_shift)

    g8 = _gather_stage(t4, s_pad, r_pad)

    scaled = _mlp_stage(g8, W1, W2, Wout, pmat)

    parts = _scatter_stage(s_pad, scaled, z_pad, g_pad, stab, shtab)
    return parts[0] + parts[1]
